# baseline (speedup 1.0000x reference)
import functools

import jax
import jax.numpy as jnp
from jax import lax
from jax.experimental import pallas as pl
from jax.experimental.pallas import tpu as pltpu
from jax.experimental.pallas import tpu_sc as plsc

B = 16384
F = 100
V = 1000
D = 64
R = 40

NC = 2
NS = 16
NW = NC * NS
BPW = B // NW
GB = 128
NBLK = BPW // GB
LANES = 16
CS = D // LANES
FP = 104
NBUF = 4


@functools.lru_cache(maxsize=1)
def _make_sc_sum():
    mesh = plsc.VectorSubcoreMesh(
        core_axis_name="c", subcore_axis_name="s", num_cores=NC, num_subcores=NS
    )

    @functools.partial(
        pl.kernel,
        out_type=jax.ShapeDtypeStruct((B, D), jnp.float32),
        mesh=mesh,
        scratch_types=[
            pltpu.VMEM((GB, FP), jnp.int32),
            [pltpu.VMEM((FP, D), jnp.float32) for _ in range(NBUF)],
            pltpu.VMEM((GB, D), jnp.float32),
            [pltpu.SemaphoreType.DMA for _ in range(NBUF)],
        ],
        compiler_params=pltpu.CompilerParams(use_tc_tiling_on_sc=False),
    )
    def _sc_sum(xp_hbm, tab_hbm, sums_hbm, idx_v, bufs, acc_v, sems):
        wid = lax.axis_index("s") * NC + lax.axis_index("c")

        def fire(r, k):
            pltpu.async_copy(tab_hbm.at[idx_v.at[r]], bufs[k], sems[k])

        def drain(k):
            pltpu.make_async_copy(
                tab_hbm.at[idx_v.at[0]], bufs[k], sems[k]
            ).wait()

        def reduce_row(r, k):
            buf = bufs[k]
            for c in range(CS):
                sl = pl.ds(c * LANES, LANES)
                acc = [buf[m, sl] for m in range(4)]
                for q in range(4, F):
                    acc[q % 4] = acc[q % 4] + buf[q, sl]
                acc_v[r, sl] = (acc[0] + acc[1]) + (acc[2] + acc[3])

        def blk_body(blk):
            base = wid * BPW + blk * GB
            pltpu.sync_copy(xp_hbm.at[pl.ds(base, GB), :], idx_v)

            for k in range(NBUF):
                fire(k, k)

            def ring_body(r):
                for k in range(NBUF):
                    drain(k)
                    reduce_row(r + k, k)

                    @pl.when(r + k + NBUF < GB)
                    def _():
                        fire(r + k + NBUF, k)

            pl.loop(0, GB, step=NBUF)(ring_body)
            pltpu.sync_copy(acc_v, sums_hbm.at[pl.ds(base, GB)])

        pl.loop(0, NBLK)(blk_body)

    return _sc_sum


def _bcast_body(s_ref, o_ref):
    st = s_ref[...].T
    o_ref[...] = jnp.broadcast_to(st[None, :, :], o_ref.shape)


_BM = 512


def _bcast(sums):
    out3 = pl.pallas_call(
        _bcast_body,
        grid=(B // _BM,),
        in_specs=[pl.BlockSpec((_BM, D), lambda i: (i, 0))],
        out_specs=pl.BlockSpec((R, D, _BM), lambda i: (0, 0, i)),
        out_shape=jax.ShapeDtypeStruct((R, D, B), jnp.float32),
    )(sums)
    return jnp.transpose(out3, (2, 0, 1))


def kernel(x, tables):
    xoff = x + (jnp.arange(F, dtype=jnp.int32) * V)[None, :]
    xp = jnp.pad(xoff, ((0, 0), (0, FP - F)))
    tab = tables.reshape(F * V, D)
    sums = _make_sc_sum()(xp, tab)
    return _bcast(sums)


# speedup vs baseline: 4.7200x; 4.7200x over previous
import functools

import jax
import jax.numpy as jnp
from jax import lax
from jax.experimental import pallas as pl
from jax.experimental.pallas import tpu as pltpu
from jax.experimental.pallas import tpu_sc as plsc

B = 16384
F = 100
V = 1000
D = 64
R = 40

NC = 2
NS = 16
NW = NC * NS
BPW = B // NW
GB = 128
NBLK = BPW // GB
LANES = 16
CS = D // LANES
FG = 5
NG = F // FG


@functools.lru_cache(maxsize=None)
def _make_sc_sum(nch=1, ci=0):
    bc = B // nch
    bpw = bc // NW
    nblk = bpw // GB
    mesh = plsc.VectorSubcoreMesh(
        core_axis_name="c", subcore_axis_name="s", num_cores=NC, num_subcores=NS
    )

    @functools.partial(
        pl.kernel,
        out_type=jax.ShapeDtypeStruct((bc, D), jnp.float32),
        mesh=mesh,
        scratch_types=[
            pltpu.VMEM((F, GB), jnp.int32),
            pltpu.VMEM((FG, GB, D), jnp.float32),
            pltpu.VMEM((FG, GB, D), jnp.float32),
            pltpu.VMEM((GB, D), jnp.float32),
            pltpu.SemaphoreType.DMA,
            pltpu.SemaphoreType.DMA,
        ],
        compiler_params=pltpu.CompilerParams(use_tc_tiling_on_sc=False),
    )
    def _sc_sum(xt_hbm, tab_hbm, sums_hbm, idx_v, buf_a, buf_b, acc_v, sem_a, sem_b):
        wid = lax.axis_index("s") * NC + lax.axis_index("c")

        def fire(g, buf, sem):
            for k in range(FG):
                pltpu.async_copy(
                    tab_hbm.at[idx_v.at[g * FG + k]], buf.at[k], sem
                )

        def drain(buf, sem):
            for k in range(FG):
                pltpu.make_async_copy(
                    tab_hbm.at[idx_v.at[k]], buf.at[k], sem
                ).wait()

        def accum(buf):
            def row_body(r):
                for c in range(CS):
                    sl = pl.ds(c * LANES, LANES)
                    s = buf[0, r, sl]
                    for k in range(1, FG):
                        s = s + buf[k, r, sl]
                    acc_v[r, sl] = acc_v[r, sl] + s

            pl.loop(0, GB)(row_body)

        def blk_body(blk):
            base = wid * bpw + blk * GB
            pltpu.sync_copy(xt_hbm.at[:, pl.ds(ci * bc + base, GB)], idx_v)

            zero = jnp.zeros((LANES,), jnp.float32)

            def zero_body(r):
                for c in range(CS):
                    acc_v[r, pl.ds(c * LANES, LANES)] = zero

            pl.loop(0, GB)(zero_body)

            fire(0, buf_a, sem_a)
            fire(1, buf_b, sem_b)

            def pair_body(g):
                drain(buf_a, sem_a)
                accum(buf_a)

                @pl.when(g + 2 < NG)
                def _():
                    fire(g + 2, buf_a, sem_a)

                drain(buf_b, sem_b)
                accum(buf_b)

                @pl.when(g + 3 < NG)
                def _():
                    fire(g + 3, buf_b, sem_b)

            pl.loop(0, NG, step=2)(pair_body)
            pltpu.sync_copy(acc_v, sums_hbm.at[pl.ds(base, GB)])

        pl.loop(0, nblk)(blk_body)

    return _sc_sum


def _bcast_body(s_ref, o_ref):
    st = s_ref[...].T
    o_ref[...] = jnp.broadcast_to(st[None, :, :], o_ref.shape)


_BM = 1024


def _bcast_chunk(sums_c, ci, nch, prev):
    bc = B // nch
    blk0 = ci * (bc // _BM)
    out_shape = jax.ShapeDtypeStruct((R, D, B), jnp.float32)
    if prev is None:
        return pl.pallas_call(
            _bcast_body,
            grid=(bc // _BM,),
            in_specs=[pl.BlockSpec((_BM, D), lambda i: (i, 0))],
            out_specs=pl.BlockSpec((R, D, _BM), lambda i: (0, 0, i + blk0)),
            out_shape=out_shape,
        )(sums_c)
    return pl.pallas_call(
        lambda p_ref, s_ref, o_ref: _bcast_body(s_ref, o_ref),
        grid=(bc // _BM,),
        in_specs=[
            pl.BlockSpec(memory_space=pl.ANY),
            pl.BlockSpec((_BM, D), lambda i: (i, 0)),
        ],
        out_specs=pl.BlockSpec((R, D, _BM), lambda i: (0, 0, i + blk0)),
        out_shape=out_shape,
        input_output_aliases={0: 0},
    )(prev, sums_c)


NCH = 1


def kernel(x, tables):
    xt = (x + (jnp.arange(F, dtype=jnp.int32) * V)[None, :]).T
    tab = tables.reshape(F * V, D)
    out3 = None
    for ci in range(NCH):
        sums_c = _make_sc_sum(NCH, ci)(xt, tab)
        out3 = _bcast_chunk(sums_c, ci, NCH, out3)
    return jnp.transpose(out3, (2, 0, 1))
